# baseline (speedup 1.0000x reference)
import jax
import jax.numpy as jnp
from jax import lax
from jax.experimental import pallas as pl
from jax.experimental.pallas import tpu as pltpu
from jax.experimental.pallas import tpu_sc as plsc

B, L = 4096, 200
V, D_IN, D_OUT = 100, 32, 16
N = B * L
VPAD = 128

NC, NS = 2, 16
NW = NC * NS
BT = B // NW
LC = 16
CHUNK_STARTS = [i * LC for i in range(12)] + [L - LC]
OUT_W = 2 * NW * 8 * 128


def _table_kernel(emb_ref, gamma_ref, beta_ref, w_ref, b_ref, out_ref):
    x = emb_ref[...]
    mean = jnp.mean(x, axis=1, keepdims=True)
    xc = x - mean
    var = jnp.mean(xc * xc, axis=1, keepdims=True)
    xn = xc * lax.rsqrt(var + 1e-5)
    xn = xn * gamma_ref[...] + beta_ref[...]
    out_ref[...] = (
        jnp.dot(xn, w_ref[...], preferred_element_type=jnp.float32)
        + b_ref[...]
    )


def _fuse_table(emb_table, ln_gamma, ln_beta, W, b):
    emb_pad = jnp.zeros((VPAD, D_IN), jnp.float32).at[:V].set(emb_table)
    return pl.pallas_call(
        _table_kernel,
        out_shape=jax.ShapeDtypeStruct((VPAD, D_OUT), jnp.float32),
    )(
        emb_pad,
        ln_gamma.reshape(1, D_IN),
        ln_beta.reshape(1, D_IN),
        W,
        b.reshape(1, D_OUT),
    )


def _sc_gather_body(
    table_hbm, ids_hbm, out_hbm,
    tab_v, idx_v, buf00, buf01, buf10, buf11, sem0, sem1,
):
    wid = lax.axis_index("s") * NC + lax.axis_index("c")
    pltpu.sync_copy(table_hbm, tab_v)
    pltpu.sync_copy(ids_hbm.at[pl.ds(wid * (BT * L), BT * L)], idx_v)

    row_iota = lax.iota(jnp.int32, 16)
    bufs = ((buf00, buf01), (buf10, buf11))
    sems = (sem0, sem1)
    handles = []
    for ci, l0 in enumerate(CHUNK_STARTS):
        par = ci % 2
        if ci >= 2:
            handles[2 * (ci - 2)].wait()
            handles[2 * (ci - 2) + 1].wait()
        buf0, buf1 = bufs[par]

        @plsc.parallel_loop(0, BT, step=1, unroll=4)
        def inner(b):
            idv = idx_v[pl.ds(b * L + l0, 16)]
            base = idv * 16
            bc = jnp.full((16,), b, jnp.int32)
            for c8 in range(8):
                colidx = bc + c8 * 128
                for ct, buf in ((0, buf0), (1, buf1)):
                    col = plsc.load_gather(tab_v, [base + (ct * 8 + c8)])
                    plsc.store_scatter(buf, [row_iota, colidx], col)

        for ct, buf in ((0, buf0), (1, buf1)):
            handles.append(
                pltpu.async_copy(
                    buf,
                    out_hbm.at[
                        pl.ds(l0, LC),
                        pl.ds(ct * (NW * 1024) + wid * 1024, 1024),
                    ],
                    sems[par],
                )
            )
    for h in handles[-4:]:
        h.wait()


def _sc_gather(table, ids_flat):
    mesh = plsc.VectorSubcoreMesh(core_axis_name="c", subcore_axis_name="s")
    run = pl.kernel(
        _sc_gather_body,
        out_type=jax.ShapeDtypeStruct((L, OUT_W), jnp.float32),
        mesh=mesh,
        scratch_types=[
            pltpu.VMEM((VPAD * D_OUT,), jnp.float32),
            pltpu.VMEM((BT * L,), jnp.int32),
            pltpu.VMEM((LC, 1024), jnp.float32),
            pltpu.VMEM((LC, 1024), jnp.float32),
            pltpu.VMEM((LC, 1024), jnp.float32),
            pltpu.VMEM((LC, 1024), jnp.float32),
            pltpu.SemaphoreType.DMA,
            pltpu.SemaphoreType.DMA,
        ],
        compiler_params=pltpu.CompilerParams(
            use_tc_tiling_on_sc=False, needs_layout_passes=False
        ),
    )
    return run(table.reshape(VPAD * D_OUT), ids_flat)


def kernel(input_ids, emb_table, ln_gamma, ln_beta, W, b):
    table = _fuse_table(emb_table, ln_gamma, ln_beta, W, b)
    ids_flat = input_ids.reshape(N).astype(jnp.int32)
    out = _sc_gather(table, ids_flat)
    out = out.reshape(L, 2, NW, 8, BT)
    out = out.transpose(2, 4, 0, 1, 3)
    return out.reshape(B, L, D_OUT)


# speedup vs baseline: 3.5387x; 3.5387x over previous
import jax
import jax.numpy as jnp
from jax import lax
from jax.experimental import pallas as pl
from jax.experimental.pallas import tpu as pltpu
from jax.experimental.pallas import tpu_sc as plsc

B, L = 4096, 200
V, D_IN, D_OUT = 100, 32, 16
N = B * L
VPAD = 128
TAB = VPAD * D_OUT

NC, NS = 2, 16
NW = NC * NS
BT = B // NW
LC = 16
CHUNK_STARTS = [i * LC for i in range(12)] + [L - LC]
OUT_W = 2 * NW * 8 * 128
S = TAB + 16
IDP = L + 1


def _table_kernel(emb_ref, gamma_ref, beta_ref, w_ref, b_ref, out_ref):
    x = emb_ref[...]
    mean = jnp.mean(x, axis=1, keepdims=True)
    xc = x - mean
    var = jnp.mean(xc * xc, axis=1, keepdims=True)
    xn = xc * lax.rsqrt(var + 1e-5)
    xn = xn * gamma_ref[...] + beta_ref[...]
    out_ref[...] = (
        jnp.dot(xn, w_ref[...], preferred_element_type=jnp.float32)
        + b_ref[...]
    )


def _fuse_table(emb_table, ln_gamma, ln_beta, W, b):
    emb_pad = jnp.zeros((VPAD, D_IN), jnp.float32).at[:V].set(emb_table)
    return pl.pallas_call(
        _table_kernel,
        out_shape=jax.ShapeDtypeStruct((VPAD, D_OUT), jnp.float32),
    )(
        emb_pad,
        ln_gamma.reshape(1, D_IN),
        ln_beta.reshape(1, D_IN),
        W,
        b.reshape(1, D_OUT),
    )


def _sc_gather_body(
    table_hbm, ids_hbm, out_hbm,
    tab_v, idx_v, buf00, buf01, buf10, buf11, sem0, sem1,
):
    wid = lax.axis_index("s") * NC + lax.axis_index("c")
    pltpu.sync_copy(table_hbm, tab_v)
    pltpu.sync_copy(
        ids_hbm.at[pl.ds(wid * BT, BT)], idx_v.at[:, pl.ds(0, L)]
    )

    iota = lax.iota(jnp.int32, 16)
    skew = iota * (S + 1)
    bufs = ((buf00, buf01), (buf10, buf11))
    sems = (sem0, sem1)
    handles = []
    for ci, l0 in enumerate(CHUNK_STARTS):
        par = ci % 2
        if ci >= 2:
            handles[2 * (ci - 2)].wait()
            handles[2 * (ci - 2) + 1].wait()
        buf0, buf1 = bufs[par]

        @plsc.parallel_loop(0, LC * 8, step=1)
        def inner(i):
            ll = lax.shift_right_logical(i, 3)
            bl0 = lax.shift_left(lax.bitwise_and(i, 7), 4)
            lv = jnp.full((16,), l0 + ll, jnp.int32)
            idv = plsc.load_gather(idx_v, [iota + bl0, lv])
            base = idv * 16
            for c in range(D_OUT):
                col = plsc.load_gather(tab_v, [base + (skew + c)])
                buf = buf0 if c < 8 else buf1
                buf[ll, pl.ds((c % 8) * 128 + bl0, 16)] = col

        for ct, buf in ((0, buf0), (1, buf1)):
            handles.append(
                pltpu.async_copy(
                    buf,
                    out_hbm.at[
                        pl.ds(l0, LC),
                        pl.ds(ct * (NW * 1024) + wid * 1024, 1024),
                    ],
                    sems[par],
                )
            )
    for h in handles[-4:]:
        h.wait()


def _sc_gather(table_skew, ids2d):
    mesh = plsc.VectorSubcoreMesh(core_axis_name="c", subcore_axis_name="s")
    run = pl.kernel(
        _sc_gather_body,
        out_type=jax.ShapeDtypeStruct((L, OUT_W), jnp.float32),
        mesh=mesh,
        scratch_types=[
            pltpu.VMEM((16 * S,), jnp.float32),
            pltpu.VMEM((BT, IDP), jnp.int32),
            pltpu.VMEM((LC, 1024), jnp.float32),
            pltpu.VMEM((LC, 1024), jnp.float32),
            pltpu.VMEM((LC, 1024), jnp.float32),
            pltpu.VMEM((LC, 1024), jnp.float32),
            pltpu.SemaphoreType.DMA,
            pltpu.SemaphoreType.DMA,
        ],
        compiler_params=pltpu.CompilerParams(
            use_tc_tiling_on_sc=False, needs_layout_passes=False
        ),
    )
    return run(table_skew, ids2d)


def kernel(input_ids, emb_table, ln_gamma, ln_beta, W, b):
    t2 = _fuse_table(emb_table, ln_gamma, ln_beta, W, b).reshape(TAB)
    table_skew = jnp.concatenate(
        [jnp.pad(t2, (k, S - TAB - k)) for k in range(16)]
    )
    ids2d = input_ids.astype(jnp.int32)
    out = _sc_gather(table_skew, ids2d)
    out = out.reshape(L, 2, NW, 8, BT)
    out = out.transpose(2, 4, 0, 1, 3)
    return out.reshape(B, L, D_OUT)


# speedup vs baseline: 3.6906x; 1.0429x over previous
import jax
import jax.numpy as jnp
from jax import lax
from jax.experimental import pallas as pl
from jax.experimental.pallas import tpu as pltpu
from jax.experimental.pallas import tpu_sc as plsc

B, L = 4096, 200
V, D_IN, D_OUT = 100, 32, 16
N = B * L
VPAD = 128
TAB = VPAD * D_OUT

NC, NS = 2, 16
NW = NC * NS
BT = B // NW
LC = 16
CHUNK_STARTS = [i * LC for i in range(12)] + [L - LC]
OUT_W = 2 * NW * 8 * 128
S = TAB + 16
IDP = L + 1


def _table_kernel(emb_ref, gamma_ref, beta_ref, w_ref, b_ref, out_ref):
    x = emb_ref[...]
    mean = jnp.mean(x, axis=1, keepdims=True)
    xc = x - mean
    var = jnp.mean(xc * xc, axis=1, keepdims=True)
    xn = xc * lax.rsqrt(var + 1e-5)
    xn = xn * gamma_ref[...] + beta_ref[...]
    out_ref[...] = (
        jnp.dot(xn, w_ref[...], preferred_element_type=jnp.float32)
        + b_ref[...]
    )


def _fuse_table(emb_table, ln_gamma, ln_beta, W, b):
    emb_pad = jnp.zeros((VPAD, D_IN), jnp.float32).at[:V].set(emb_table)
    return pl.pallas_call(
        _table_kernel,
        out_shape=jax.ShapeDtypeStruct((VPAD, D_OUT), jnp.float32),
    )(
        emb_pad,
        ln_gamma.reshape(1, D_IN),
        ln_beta.reshape(1, D_IN),
        W,
        b.reshape(1, D_OUT),
    )


def _sc_gather_body(
    table_hbm, ids_hbm, out_hbm,
    tab_v, idx_v, buf00, buf01, buf10, buf11, sem0, sem1,
):
    wid = lax.axis_index("s") * NC + lax.axis_index("c")
    pltpu.sync_copy(table_hbm, tab_v)
    pltpu.sync_copy(
        ids_hbm.at[pl.ds(wid * BT, BT)], idx_v.at[:, pl.ds(0, L)]
    )

    iota = lax.iota(jnp.int32, 16)
    skew = iota * (S + 1)
    bufs = ((buf00, buf01), (buf10, buf11))
    sems = (sem0, sem1)
    n_chunks = len(CHUNK_STARTS)

    def chunk_body(ci, carry):
        par = lax.rem(ci, 2)
        l0 = jnp.minimum(ci * LC, L - LC)

        for p in (0, 1):
            @pl.when((par == p) & (ci >= 2))
            def _():
                for buf in bufs[p]:
                    pltpu.make_async_copy(
                        out_hbm.at[pl.ds(0, LC), pl.ds(0, 1024)], buf, sems[p]
                    ).wait()

        for p in (0, 1):
            @pl.when(par == p)
            def _():
                buf0, buf1 = bufs[p]

                @plsc.parallel_loop(0, LC * 8, step=1)
                def inner(i):
                    ll = lax.shift_right_logical(i, 3)
                    bl0 = lax.shift_left(lax.bitwise_and(i, 7), 4)
                    lv = jnp.full((16,), l0 + ll, jnp.int32)
                    idv = plsc.load_gather(idx_v, [iota + bl0, lv])
                    base = idv * 16
                    for c in range(D_OUT):
                        col = plsc.load_gather(tab_v, [base + (skew + c)])
                        buf = buf0 if c < 8 else buf1
                        buf[ll, pl.ds((c % 8) * 128 + bl0, 16)] = col

                for ct, buf in ((0, buf0), (1, buf1)):
                    pltpu.async_copy(
                        buf,
                        out_hbm.at[
                            pl.ds(l0, LC),
                            pl.ds(ct * (NW * 1024) + wid * 1024, 1024),
                        ],
                        sems[p],
                    )
        return carry

    lax.fori_loop(0, n_chunks, chunk_body, 0)
    for par in (0, 1):
        for buf in bufs[par]:
            pltpu.make_async_copy(
                out_hbm.at[pl.ds(0, LC), pl.ds(0, 1024)], buf, sems[par]
            ).wait()


def _sc_gather(table_skew, ids2d):
    mesh = plsc.VectorSubcoreMesh(core_axis_name="c", subcore_axis_name="s")
    run = pl.kernel(
        _sc_gather_body,
        out_type=jax.ShapeDtypeStruct((L, OUT_W), jnp.float32),
        mesh=mesh,
        scratch_types=[
            pltpu.VMEM((16 * S,), jnp.float32),
            pltpu.VMEM((BT, IDP), jnp.int32),
            pltpu.VMEM((LC, 1024), jnp.float32),
            pltpu.VMEM((LC, 1024), jnp.float32),
            pltpu.VMEM((LC, 1024), jnp.float32),
            pltpu.VMEM((LC, 1024), jnp.float32),
            pltpu.SemaphoreType.DMA,
            pltpu.SemaphoreType.DMA,
        ],
        compiler_params=pltpu.CompilerParams(
            use_tc_tiling_on_sc=False, needs_layout_passes=False
        ),
    )
    return run(table_skew, ids2d)


def kernel(input_ids, emb_table, ln_gamma, ln_beta, W, b):
    t2 = _fuse_table(emb_table, ln_gamma, ln_beta, W, b).reshape(TAB)
    table_skew = jnp.concatenate(
        [jnp.pad(t2, (k, S - TAB - k)) for k in range(16)]
    )
    ids2d = input_ids.astype(jnp.int32)
    out = _sc_gather(table_skew, ids2d)
    out = out.reshape(L, 2, NW, 8, BT)
    out = out.transpose(2, 4, 0, 1, 3)
    return out.reshape(B, L, D_OUT)
